# Initial kernel scaffold; baseline (speedup 1.0000x reference)
import jax
import jax.numpy as jnp
from jax.experimental import pallas as pl


def kernel(state, feature, edge_dist, W1, b1, W2, b2, W3, b3, w_scalar, src, dst):
    raise NotImplementedError("write your pallas kernel here")


# speedup vs baseline: 4.6058x; 4.6058x over previous
import functools

import jax
import jax.numpy as jnp
import numpy as np
from jax import lax
from jax.experimental import pallas as pl
from jax.experimental.pallas import tpu as pltpu
from jax.experimental.pallas import tpu_sc as plsc

N_NODES = 10000
N_EDGES = 320000
H = 8
F = 40
DD = 16

NW = 32
CH = 128
K = 79
EW = K * CH
E_PAD = NW * EW
N_PAD = 10240
NBLK = 256
EBLK = 512


def _prep_body(f_ref, s_ref, w1a_ref, w1b_ref, tsrc_ref, tdst_ref):
    f = f_ref[...]
    s = s_ref[...]
    a = jnp.dot(f, w1a_ref[...], preferred_element_type=jnp.float32)
    b = jnp.dot(f, w1b_ref[...], preferred_element_type=jnp.float32)
    z = jnp.zeros((f.shape[0], 8), jnp.float32)
    tsrc_ref[...] = jnp.concatenate([a, s, z], axis=1)
    tdst_ref[...] = jnp.concatenate([b, s, z], axis=1)


def _prep(feature_p, state_p, w1a, w1b):
    grid = (N_PAD // NBLK,)
    return pl.pallas_call(
        _prep_body,
        grid=grid,
        in_specs=[
            pl.BlockSpec((NBLK, F), lambda i: (i, 0)),
            pl.BlockSpec((NBLK, H), lambda i: (i, 0)),
            pl.BlockSpec((F, 16), lambda i: (0, 0)),
            pl.BlockSpec((F, 16), lambda i: (0, 0)),
        ],
        out_specs=[
            pl.BlockSpec((NBLK, 32), lambda i: (i, 0)),
            pl.BlockSpec((NBLK, 32), lambda i: (i, 0)),
        ],
        out_shape=[
            jax.ShapeDtypeStruct((N_PAD, 32), jnp.float32),
            jax.ShapeDtypeStruct((N_PAD, 32), jnp.float32),
        ],
    )(feature_p, state_p, w1a, w1b)


def _make_gather():
    mesh = plsc.VectorSubcoreMesh(core_axis_name="c", subcore_axis_name="s")

    @functools.partial(
        pl.kernel,
        mesh=mesh,
        compiler_params=pltpu.CompilerParams(use_tc_tiling_on_sc=False),
        out_type=[
            jax.ShapeDtypeStruct((E_PAD, 32), jnp.float32),
            jax.ShapeDtypeStruct((E_PAD, 32), jnp.float32),
        ],
        scratch_types=[
            pltpu.VMEM((CH,), jnp.int32),
            pltpu.VMEM((CH,), jnp.int32),
            pltpu.VMEM((CH, 32), jnp.float32),
            pltpu.VMEM((CH, 32), jnp.float32),
            pltpu.SemaphoreType.DMA,
            pltpu.SemaphoreType.DMA,
        ],
    )
    def gather_k(tsrc, tdst, src, dst, gsrc, gdst, idxs, idxd, rs, rd, sem1, sem2):
        cid = lax.axis_index("c")
        sid = lax.axis_index("s")
        wid = sid * 2 + cid
        base = wid * EW

        def body(j, carry):
            off = base + j * CH
            pltpu.sync_copy(src.at[pl.ds(off, CH)], idxs)
            pltpu.sync_copy(dst.at[pl.ds(off, CH)], idxd)
            ca = pltpu.async_copy(tsrc.at[idxs], rs, sem1)
            cb = pltpu.async_copy(tdst.at[idxd], rd, sem2)
            ca.wait()
            cb.wait()
            pltpu.sync_copy(rs, gsrc.at[pl.ds(off, CH)])
            pltpu.sync_copy(rd, gdst.at[pl.ds(off, CH)])
            return carry

        lax.fori_loop(0, K, body, 0)

    return gather_k


def _mlp_body(gs_ref, gd_ref, ed_ref, w1c_ref, b1_ref, w2_ref, b2_ref,
              w3_ref, b3_ref, rep_ref, summ_ref, out_ref):
    gs = gs_ref[...]
    gd = gd_ref[...]
    asrc = gs[:, :16]
    ssrc = gs[:, 16:24]
    bdst = gd[:, :16]
    sdst = gd[:, 16:24]
    pre1 = asrc + bdst + jnp.dot(ed_ref[...], w1c_ref[...],
                                 preferred_element_type=jnp.float32) + b1_ref[...]
    h1 = jax.nn.sigmoid(pre1)
    h2 = jax.nn.sigmoid(jnp.dot(h1, w2_ref[...],
                                preferred_element_type=jnp.float32) + b2_ref[...])
    w = jax.nn.sigmoid(jnp.dot(h2, w3_ref[...],
                               preferred_element_type=jnp.float32) + b3_ref[...])
    cat = jnp.concatenate([ssrc, sdst], axis=1)
    catx = jnp.dot(cat, rep_ref[...], preferred_element_type=jnp.float32)
    r = jnp.dot(catx * w, summ_ref[...], preferred_element_type=jnp.float32)
    alpha = jnp.where(r >= 0, r, 0.01 * r)
    ex = jnp.exp(alpha)
    out_ref[...] = jnp.concatenate([ex, ex * ssrc], axis=1)


def _mlp(gsrc, gdst, ed_p, w1c, b1, w2, b2, w3, b3, rep, summ):
    grid = (E_PAD // EBLK,)
    full = lambda shape: pl.BlockSpec(shape, lambda i: (0, 0))
    return pl.pallas_call(
        _mlp_body,
        grid=grid,
        in_specs=[
            pl.BlockSpec((EBLK, 32), lambda i: (i, 0)),
            pl.BlockSpec((EBLK, 32), lambda i: (i, 0)),
            pl.BlockSpec((EBLK, DD), lambda i: (i, 0)),
            full((DD, 16)),
            full((1, 16)),
            full((16, 2)),
            full((1, 2)),
            full((2, 128)),
            full((1, 128)),
            full((16, 128)),
            full((128, 8)),
        ],
        out_specs=pl.BlockSpec((EBLK, 16), lambda i: (i, 0)),
        out_shape=jax.ShapeDtypeStruct((E_PAD, 16), jnp.float32),
    )(gsrc, gdst, ed_p, w1c, b1, w2, b2, w3, b3, rep, summ)


def _make_scatter():
    mesh = plsc.VectorSubcoreMesh(core_axis_name="c", subcore_axis_name="s")
    stripe = N_PAD // 16

    @functools.partial(
        pl.kernel,
        mesh=mesh,
        compiler_params=pltpu.CompilerParams(use_tc_tiling_on_sc=False),
        out_type=jax.ShapeDtypeStruct((2, N_PAD, 16), jnp.float32),
        scratch_types=[
            pltpu.VMEM_SHARED((N_PAD, 16), jnp.float32),
            pltpu.VMEM((CH,), jnp.int32),
            pltpu.VMEM((CH, 16), jnp.float32),
        ],
    )
    def scatter_k(exv, dst, zeros, partials, acc, idxd, val):
        cid = lax.axis_index("c")
        sid = lax.axis_index("s")
        wid = sid * 2 + cid
        base = wid * EW
        pltpu.sync_copy(zeros.at[pl.ds(sid * stripe, stripe)],
                        acc.at[pl.ds(sid * stripe, stripe)])
        plsc.subcore_barrier()

        def body(j, carry):
            off = base + j * CH
            pltpu.sync_copy(dst.at[pl.ds(off, CH)], idxd)
            pltpu.sync_copy(exv.at[pl.ds(off, CH)], val)
            pltpu.sync_copy(val, acc.at[idxd], add=True)
            return carry

        lax.fori_loop(0, K, body, 0)
        plsc.subcore_barrier()

        @pl.when(sid == 0)
        def _():
            pltpu.sync_copy(acc, partials.at[cid])

    return scatter_k


def _fin_body(p_ref, ws_ref, out_ref):
    p = p_ref[...]
    d = p[0, :, :8] + p[1, :, :8]
    nu = p[0, :, 8:] + p[1, :, 8:]
    b = jax.nn.sigmoid(ws_ref[...])
    a = nu / (d + 1e-9)
    out_ref[...] = jnp.maximum(a * b, 0.0)


def _finalize(partials, ws):
    grid = (N_PAD // NBLK,)
    return pl.pallas_call(
        _fin_body,
        grid=grid,
        in_specs=[
            pl.BlockSpec((2, NBLK, 16), lambda i: (0, i, 0)),
            pl.BlockSpec((1, 1), lambda i: (0, 0)),
        ],
        out_specs=pl.BlockSpec((NBLK, H), lambda i: (i, 0)),
        out_shape=jax.ShapeDtypeStruct((N_PAD, H), jnp.float32),
    )(partials, ws)


def kernel(state, feature, edge_dist, W1, b1, W2, b2, W3, b3, w_scalar, src, dst):
    state_p = jnp.pad(state, ((0, N_PAD - N_NODES), (0, 0)))
    feature_p = jnp.pad(feature, ((0, N_PAD - N_NODES), (0, 0)))
    src_p = jnp.pad(src, (0, E_PAD - N_EDGES))
    dst_p = jnp.pad(dst, (0, E_PAD - N_EDGES), constant_values=N_NODES)
    ed_p = jnp.pad(edge_dist, ((0, E_PAD - N_EDGES), (0, 0)))

    w1a = W1[:F]
    w1b = W1[F:2 * F]
    w1c = W1[2 * F:]
    rep = jnp.asarray(np.kron(np.eye(16, dtype=np.float32), np.ones((1, 8), np.float32)))
    summ = jnp.asarray(np.tile(np.eye(8, dtype=np.float32), (16, 1)))
    zeros = jnp.zeros((N_PAD, 16), jnp.float32)

    tsrc, tdst = _prep(feature_p, state_p, w1a, w1b)
    gsrc, gdst = _make_gather()(tsrc, tdst, src_p, dst_p)
    exv = _mlp(gsrc, gdst, ed_p, w1c, b1.reshape(1, 16), W2, b2.reshape(1, 2),
               W3, b3.reshape(1, 128), rep, summ)
    partials = _make_scatter()(exv, dst_p, zeros)
    out_p = _finalize(partials, w_scalar.reshape(1, 1))
    return out_p[:N_NODES]


# speedup vs baseline: 6.8645x; 1.4904x over previous
import functools

import jax
import jax.numpy as jnp
import numpy as np
from jax import lax
from jax.experimental import pallas as pl
from jax.experimental.pallas import tpu as pltpu
from jax.experimental.pallas import tpu_sc as plsc

N_NODES = 10000
N_EDGES = 320000
H = 8
F = 40
DD = 16

NW = 32
CH = 128
K = 79
EW = K * CH
E_PAD = NW * EW
N_PAD = 10240
NBLK = 256
EBLK = 2048


def _prep_body(f_ref, s_ref, w1a_ref, w1b_ref, tsrc_ref, tdst_ref):
    f = f_ref[...]
    s = s_ref[...]
    a = jnp.dot(f, w1a_ref[...], preferred_element_type=jnp.float32)
    b = jnp.dot(f, w1b_ref[...], preferred_element_type=jnp.float32)
    z = jnp.zeros((f.shape[0], 8), jnp.float32)
    tsrc_ref[...] = jnp.concatenate([a, s, z], axis=1)
    tdst_ref[...] = jnp.concatenate([b, s, z], axis=1)


def _prep(feature_p, state_p, w1a, w1b):
    grid = (N_PAD // NBLK,)
    return pl.pallas_call(
        _prep_body,
        grid=grid,
        in_specs=[
            pl.BlockSpec((NBLK, F), lambda i: (i, 0)),
            pl.BlockSpec((NBLK, H), lambda i: (i, 0)),
            pl.BlockSpec((F, 16), lambda i: (0, 0)),
            pl.BlockSpec((F, 16), lambda i: (0, 0)),
        ],
        out_specs=[
            pl.BlockSpec((NBLK, 32), lambda i: (i, 0)),
            pl.BlockSpec((NBLK, 32), lambda i: (i, 0)),
        ],
        out_shape=[
            jax.ShapeDtypeStruct((N_PAD, 32), jnp.float32),
            jax.ShapeDtypeStruct((N_PAD, 32), jnp.float32),
        ],
    )(feature_p, state_p, w1a, w1b)


def _make_gather():
    mesh = plsc.VectorSubcoreMesh(core_axis_name="c", subcore_axis_name="s")

    @functools.partial(
        pl.kernel,
        mesh=mesh,
        compiler_params=pltpu.CompilerParams(use_tc_tiling_on_sc=False),
        out_type=[
            jax.ShapeDtypeStruct((E_PAD, 32), jnp.float32),
            jax.ShapeDtypeStruct((E_PAD, 32), jnp.float32),
        ],
        scratch_types=[
            pltpu.VMEM((K, CH), jnp.int32),
            pltpu.VMEM((K, CH), jnp.int32),
            pltpu.VMEM((2, CH, 32), jnp.float32),
            pltpu.VMEM((2, CH, 32), jnp.float32),
            pltpu.SemaphoreType.DMA,
            pltpu.SemaphoreType.DMA,
            pltpu.SemaphoreType.DMA,
            pltpu.SemaphoreType.DMA,
        ],
    )
    def gather_k(tsrc, tdst, src2d, dst2d, gsrc, gdst,
                 idxs, idxd, rs, rd, sg0, sg1, sw0, sw1):
        cid = lax.axis_index("c")
        sid = lax.axis_index("s")
        wid = sid * 2 + cid
        base = wid * EW

        pltpu.sync_copy(src2d.at[pl.ds(wid * K, K)], idxs)
        pltpu.sync_copy(dst2d.at[pl.ds(wid * K, K)], idxd)

        def start_gather(j, b):
            pltpu.async_copy(tsrc.at[idxs.at[j]], rs.at[b], sg0 if b == 0 else sg1)
            pltpu.async_copy(tdst.at[idxd.at[j]], rd.at[b], sg0 if b == 0 else sg1)

        def wait_gather(b):
            sem = sg0 if b == 0 else sg1
            pltpu.make_async_copy(tsrc.at[idxs.at[0]], rs.at[b], sem).wait()
            pltpu.make_async_copy(tdst.at[idxd.at[0]], rd.at[b], sem).wait()

        def start_write(j, b):
            off = base + j * CH
            pltpu.async_copy(rs.at[b], gsrc.at[pl.ds(off, CH)], sw0 if b == 0 else sw1)
            pltpu.async_copy(rd.at[b], gdst.at[pl.ds(off, CH)], sw0 if b == 0 else sw1)

        def wait_write(b):
            sem = sw0 if b == 0 else sw1
            pltpu.make_async_copy(rs.at[b], gsrc.at[pl.ds(base, CH)], sem).wait()
            pltpu.make_async_copy(rd.at[b], gdst.at[pl.ds(base, CH)], sem).wait()

        start_gather(0, 0)

        def body(j, carry):
            b = lax.rem(j, 2)
            nb = 1 - b

            @pl.when(j >= 1)
            def _():
                @pl.when(nb == 0)
                def _():
                    wait_write(0)

                @pl.when(nb == 1)
                def _():
                    wait_write(1)

            @pl.when(nb == 0)
            def _():
                start_gather(j + 1, 0)

            @pl.when(nb == 1)
            def _():
                start_gather(j + 1, 1)

            @pl.when(b == 0)
            def _():
                wait_gather(0)
                start_write(j, 0)

            @pl.when(b == 1)
            def _():
                wait_gather(1)
                start_write(j, 1)

            return carry

        lax.fori_loop(0, K - 1, body, 0)
        last = K - 1
        lb = last % 2
        wait_gather(lb)
        start_write(last, lb)
        wait_write(1 - lb)
        wait_write(lb)

    return gather_k


def _mlp_body(gs_ref, gd_ref, ed_ref, w1c_ref, b1_ref, w2_ref, b2_ref,
              w3_ref, b3_ref, rep_ref, summ_ref, out_ref):
    gs = gs_ref[...]
    gd = gd_ref[...]
    asrc = gs[:, :16]
    ssrc = gs[:, 16:24]
    bdst = gd[:, :16]
    sdst = gd[:, 16:24]
    pre1 = asrc + bdst + jnp.dot(ed_ref[...], w1c_ref[...],
                                 preferred_element_type=jnp.float32) + b1_ref[...]
    h1 = jax.nn.sigmoid(pre1)
    h2 = jax.nn.sigmoid(jnp.dot(h1, w2_ref[...],
                                preferred_element_type=jnp.float32) + b2_ref[...])
    w = jax.nn.sigmoid(jnp.dot(h2, w3_ref[...],
                               preferred_element_type=jnp.float32) + b3_ref[...])
    cat = jnp.concatenate([ssrc, sdst], axis=1)
    catx = jnp.dot(cat, rep_ref[...], preferred_element_type=jnp.float32)
    r = jnp.dot(catx * w, summ_ref[...], preferred_element_type=jnp.float32)
    alpha = jnp.where(r >= 0, r, 0.01 * r)
    ex = jnp.exp(alpha)
    out_ref[...] = jnp.concatenate([ex, ex * ssrc], axis=1)


def _mlp(gsrc, gdst, ed_p, w1c, b1, w2, b2, w3, b3, rep, summ):
    grid = (E_PAD // EBLK,)
    full = lambda shape: pl.BlockSpec(shape, lambda i: (0, 0))
    return pl.pallas_call(
        _mlp_body,
        grid=grid,
        in_specs=[
            pl.BlockSpec((EBLK, 32), lambda i: (i, 0)),
            pl.BlockSpec((EBLK, 32), lambda i: (i, 0)),
            pl.BlockSpec((EBLK, DD), lambda i: (i, 0)),
            full((DD, 16)),
            full((1, 16)),
            full((16, 2)),
            full((1, 2)),
            full((2, 128)),
            full((1, 128)),
            full((16, 128)),
            full((128, 8)),
        ],
        out_specs=pl.BlockSpec((EBLK, 16), lambda i: (i, 0)),
        out_shape=jax.ShapeDtypeStruct((E_PAD, 16), jnp.float32),
    )(gsrc, gdst, ed_p, w1c, b1, w2, b2, w3, b3, rep, summ)


def _make_scatter():
    mesh = plsc.VectorSubcoreMesh(core_axis_name="c", subcore_axis_name="s")
    stripe = N_PAD // 16

    @functools.partial(
        pl.kernel,
        mesh=mesh,
        compiler_params=pltpu.CompilerParams(use_tc_tiling_on_sc=False),
        out_type=jax.ShapeDtypeStruct((2, N_PAD, 16), jnp.float32),
        scratch_types=[
            pltpu.VMEM_SHARED((N_PAD, 16), jnp.float32),
            pltpu.VMEM((K, CH), jnp.int32),
            pltpu.VMEM((2, CH, 16), jnp.float32),
            pltpu.SemaphoreType.DMA,
            pltpu.SemaphoreType.DMA,
            pltpu.SemaphoreType.DMA,
            pltpu.SemaphoreType.DMA,
        ],
    )
    def scatter_k(exv, dst2d, zeros, partials, acc, idxd, val, sv0, sv1, sa0, sa1):
        cid = lax.axis_index("c")
        sid = lax.axis_index("s")
        wid = sid * 2 + cid
        base = wid * EW
        pltpu.sync_copy(zeros.at[pl.ds(sid * stripe, stripe)],
                        acc.at[pl.ds(sid * stripe, stripe)])
        pltpu.sync_copy(dst2d.at[pl.ds(wid * K, K)], idxd)
        plsc.subcore_barrier()

        def start_load(j, b):
            off = base + j * CH
            pltpu.async_copy(exv.at[pl.ds(off, CH)], val.at[b], sv0 if b == 0 else sv1)

        def wait_load(b):
            pltpu.make_async_copy(exv.at[pl.ds(base, CH)], val.at[b],
                                  sv0 if b == 0 else sv1).wait()

        def start_add(j, b):
            pltpu.async_copy(val.at[b], acc.at[idxd.at[j]], sa0 if b == 0 else sa1,
                             add=True)

        def wait_add(b):
            pltpu.make_async_copy(val.at[b], acc.at[idxd.at[0]],
                                  sa0 if b == 0 else sa1).wait()

        start_load(0, 0)

        def body(j, carry):
            b = lax.rem(j, 2)
            nb = 1 - b

            @pl.when(j >= 1)
            def _():
                @pl.when(nb == 0)
                def _():
                    wait_add(0)

                @pl.when(nb == 1)
                def _():
                    wait_add(1)

            @pl.when(nb == 0)
            def _():
                start_load(j + 1, 0)

            @pl.when(nb == 1)
            def _():
                start_load(j + 1, 1)

            @pl.when(b == 0)
            def _():
                wait_load(0)
                start_add(j, 0)

            @pl.when(b == 1)
            def _():
                wait_load(1)
                start_add(j, 1)

            return carry

        lax.fori_loop(0, K - 1, body, 0)
        last = K - 1
        lb = last % 2
        wait_load(lb)
        start_add(last, lb)
        wait_add(1 - lb)
        wait_add(lb)
        plsc.subcore_barrier()

        @pl.when(sid == 0)
        def _():
            pltpu.sync_copy(acc, partials.at[cid])

    return scatter_k


def _fin_body(p_ref, ws_ref, out_ref):
    p = p_ref[...]
    d = p[0, :, :8] + p[1, :, :8]
    nu = p[0, :, 8:] + p[1, :, 8:]
    b = jax.nn.sigmoid(ws_ref[...])
    a = nu / (d + 1e-9)
    out_ref[...] = jnp.maximum(a * b, 0.0)


def _finalize(partials, ws):
    grid = (N_PAD // NBLK,)
    return pl.pallas_call(
        _fin_body,
        grid=grid,
        in_specs=[
            pl.BlockSpec((2, NBLK, 16), lambda i: (0, i, 0)),
            pl.BlockSpec((1, 1), lambda i: (0, 0)),
        ],
        out_specs=pl.BlockSpec((NBLK, H), lambda i: (i, 0)),
        out_shape=jax.ShapeDtypeStruct((N_PAD, H), jnp.float32),
    )(partials, ws)


def kernel(state, feature, edge_dist, W1, b1, W2, b2, W3, b3, w_scalar, src, dst):
    state_p = jnp.pad(state, ((0, N_PAD - N_NODES), (0, 0)))
    feature_p = jnp.pad(feature, ((0, N_PAD - N_NODES), (0, 0)))
    src_p = jnp.pad(src, (0, E_PAD - N_EDGES))
    dst_p = jnp.pad(dst, (0, E_PAD - N_EDGES), constant_values=N_NODES)
    ed_p = jnp.pad(edge_dist, ((0, E_PAD - N_EDGES), (0, 0)))

    w1a = W1[:F]
    w1b = W1[F:2 * F]
    w1c = W1[2 * F:]
    rep = jnp.asarray(np.kron(np.eye(16, dtype=np.float32), np.ones((1, 8), np.float32)))
    summ = jnp.asarray(np.tile(np.eye(8, dtype=np.float32), (16, 1)))
    zeros = jnp.zeros((N_PAD, 16), jnp.float32)

    src2d = src_p.reshape(NW * K, CH)
    dst2d = dst_p.reshape(NW * K, CH)
    tsrc, tdst = _prep(feature_p, state_p, w1a, w1b)
    gsrc, gdst = _make_gather()(tsrc, tdst, src2d, dst2d)
    exv = _mlp(gsrc, gdst, ed_p, w1c, b1.reshape(1, 16), W2, b2.reshape(1, 2),
               W3, b3.reshape(1, 128), rep, summ)
    partials = _make_scatter()(exv, dst2d, zeros)
    out_p = _finalize(partials, w_scalar.reshape(1, 1))
    return out_p[:N_NODES]
